# baseline (speedup 1.0000x reference)
import functools

import jax
import jax.numpy as jnp
from jax.experimental import pallas as pl
from jax.experimental.pallas import tpu as pltpu

_BN = 3200


def _fused_kernel(tok_ref, emb_ref, W1_ref, b1_ref, W2_ref, b2_ref,
                  ntpW_ref, ntpb_ref, out_ref, v_ref, acc_ref, sem,
                  *, nexp, nsplit):
    i = pl.program_id(0)
    nchain = nsplit * nexp

    @pl.when(i == 0)
    def _gather():
        tok = tok_ref[0, tok_ref.shape[1] - 1]
        cp = pltpu.make_async_copy(
            emb_ref.at[pl.ds(tok, 1), :], v_ref, sem)
        cp.start()
        cp.wait()

    @pl.when(i < nchain)
    def _expert_piece():
        e = i // nsplit
        q = i % nsplit
        b1q = b1_ref[pl.ds(i + nsplit, 1), :]
        t = jnp.maximum(
            jnp.dot(v_ref[...], W1_ref[0],
                    preferred_element_type=jnp.float32) + b1q, 0.0)
        part = jnp.dot(t, W2_ref[0], preferred_element_type=jnp.float32)

        @pl.when(q == 0)
        def _():
            acc_ref[...] = part

        @pl.when(q != 0)
        def _():
            acc_ref[...] += part

        @pl.when(q == nsplit - 1)
        def _():
            v_ref[...] = acc_ref[...] + b2_ref[pl.ds(e + 1, 1), :]

    @pl.when(i >= nchain)
    def _project():
        out_ref[...] = (
            jnp.dot(v_ref[...], ntpW_ref[...],
                    preferred_element_type=jnp.float32) + ntpb_ref[...])


def kernel(X, emb, W1, b1, W2, b2, ntp_W, ntp_b):
    vocab, d = emb.shape
    nblocks, _, dff = W1.shape
    nexp = nblocks - 1
    nsplit = 4
    bf = dff // nsplit
    nchain = nsplit * nexp
    nv = vocab // _BN

    tok = X.astype(jnp.int32)
    b1r = b1.reshape(nblocks * nsplit, bf)

    def _e(i):
        return jnp.minimum(i // nsplit, nexp - 1) + 1

    def _h(i):
        return jnp.minimum(i, nchain - 1) % nsplit

    def _j(i):
        return jnp.maximum(i - nchain, 0)

    body = functools.partial(_fused_kernel, nexp=nexp, nsplit=nsplit)
    logits = pl.pallas_call(
        body,
        grid=(nchain + nv,),
        in_specs=[
            pl.BlockSpec(memory_space=pltpu.SMEM),
            pl.BlockSpec(memory_space=pl.ANY),
            pl.BlockSpec((1, d, bf), lambda i: (_e(i), 0, _h(i))),
            pl.BlockSpec((nblocks * nsplit, bf), lambda i: (0, 0)),
            pl.BlockSpec((1, bf, d), lambda i: (_e(i), _h(i), 0)),
            pl.BlockSpec((nblocks, d), lambda i: (0, 0)),
            pl.BlockSpec((d, _BN), lambda i: (0, _j(i))),
            pl.BlockSpec((1, _BN), lambda i: (0, _j(i))),
        ],
        out_specs=pl.BlockSpec((1, _BN), lambda i: (0, _j(i))),
        out_shape=jax.ShapeDtypeStruct((1, vocab), jnp.float32),
        scratch_shapes=[pltpu.VMEM((1, d), jnp.float32),
                        pltpu.VMEM((1, d), jnp.float32),
                        pltpu.SemaphoreType.DMA],
    )(tok, emb, W1, b1r, W2, b2, ntp_W, ntp_b.reshape(1, vocab))
    return logits
